# baseline (speedup 1.0000x reference)
import functools

import jax
import jax.numpy as jnp
from jax import lax
from jax.experimental import pallas as pl
from jax.experimental.pallas import tpu as pltpu
from jax.experimental.pallas import tpu_sc as plsc

N = 10000
D = 128
DH = 64
CW = 16
E = 320000
NC = 2
NS = 16
CHUNK = 128
NCH = 157
EPAD = NS * NCH * CHUNK
NPAD = 10240
RPS = NPAD // NS
NBUF = 4
NGRP = NCH // NBUF


@functools.partial(
    pl.kernel,
    out_type=jax.ShapeDtypeStruct((NC, NPAD, DH + CW), jnp.float32),
    mesh=plsc.VectorSubcoreMesh(core_axis_name="c", subcore_axis_name="s"),
    compiler_params=pltpu.CompilerParams(use_tc_tiling_on_sc=False),
    scratch_types=[
        pltpu.VMEM((NCH, CHUNK), jnp.int32),
        pltpu.VMEM((NCH, CHUNK), jnp.int32),
        pltpu.VMEM((NBUF, CHUNK, DH), jnp.float32),
        pltpu.VMEM((CHUNK, CW), jnp.float32),
        pltpu.VMEM((CHUNK, CW), jnp.float32),
        pltpu.VMEM_SHARED((NPAD, DH), jnp.float32),
        pltpu.VMEM_SHARED((NPAD, CW), jnp.float32),
        [pltpu.SemaphoreType.DMA] * NBUF,
        [pltpu.SemaphoreType.DMA] * NBUF,
        pltpu.SemaphoreType.DMA,
    ],
)
def _sc_aggregate(srcidx, dstidx, xflat, out_fc, src_v, dst_v, ring,
                  ones_v, zero_v, acc_f, acc_c, gsems, ssems, csem):
    cid = lax.axis_index("c")
    sid = lax.axis_index("s")

    zv = jnp.zeros((16,), jnp.float32)
    ov = jnp.ones((16,), jnp.float32)

    def zrow(i, carry):
        for k in range(DH // 16):
            ring[0, i, pl.ds(k * 16, 16)] = zv
        ones_v[i, pl.ds(0, 16)] = ov
        zero_v[i, pl.ds(0, 16)] = zv
        return carry

    lax.fori_loop(0, CHUNK, zrow, 0)
    for k in range(RPS // CHUNK):
        pltpu.sync_copy(ring.at[0],
                        acc_f.at[pl.ds(sid * RPS + k * CHUNK, CHUNK)])
        pltpu.sync_copy(zero_v,
                        acc_c.at[pl.ds(sid * RPS + k * CHUNK, CHUNK)])
    plsc.subcore_barrier()

    pltpu.sync_copy(srcidx.at[sid], src_v)
    pltpu.sync_copy(dstidx.at[sid], dst_v)
    xp = xflat.at[pl.ds(cid, 2 * N - 1)]

    def group(g, carry):
        j0 = g * NBUF
        gd = [None] * NBUF
        for k in range(NBUF):
            @pl.when(g > 0)
            def _(k=k):
                pltpu.make_async_copy(ring.at[k], acc_f.at[pl.ds(0, CHUNK)],
                                      ssems[k]).wait()

            jj = jnp.minimum(j0 + k, NCH - 1)
            gd[k] = pltpu.async_copy(xp.at[src_v.at[jj]],
                                     ring.at[k], gsems[k])
        for k in range(NBUF):
            gd[k].wait()

            @pl.when(g < NGRP)
            def _(k=k):
                pltpu.async_copy(ring.at[k], acc_f.at[dst_v.at[j0 + k]],
                                 ssems[k], add=True)

            @pl.when(cid == (k % 2))
            def _(k=k):
                @pl.when(g > 0)
                def _():
                    pltpu.make_async_copy(ones_v, acc_c.at[pl.ds(0, CHUNK)],
                                          csem).wait()

                @pl.when(g < NGRP)
                def _():
                    pltpu.async_copy(ones_v, acc_c.at[dst_v.at[j0 + k]],
                                     csem, add=True)
        return carry

    lax.fori_loop(0, NGRP + 1, group, 0)

    ntail = NCH - NGRP * NBUF
    j0 = NGRP * NBUF
    for k in range(ntail):
        g = pltpu.async_copy(xp.at[src_v.at[j0 + k]], ring.at[k],
                             gsems[k])
        g.wait()
        pltpu.async_copy(ring.at[k], acc_f.at[dst_v.at[j0 + k]],
                         ssems[k], add=True).wait()

        @pl.when(cid == (k % 2))
        def _(k=k):
            pltpu.async_copy(ones_v, acc_c.at[dst_v.at[j0 + k]], csem,
                             add=True).wait()

    plsc.subcore_barrier()
    pltpu.sync_copy(acc_f.at[pl.ds(sid * RPS, RPS)],
                    out_fc.at[cid, pl.ds(sid * RPS, RPS), pl.ds(0, DH)])
    pltpu.sync_copy(acc_c.at[pl.ds(sid * RPS, RPS)],
                    out_fc.at[cid, pl.ds(sid * RPS, RPS), pl.ds(DH, CW)])


def _tc_body(a0, a1, xr, wl_lo, wl_hi, wr, wo, bl, bo, out):
    cnt = a0[0, :, DH:DH + 1] + a1[0, :, DH:DH + 1]
    inv = 1.0 / jnp.maximum(cnt, 1.0)
    mean_l = a0[0, :, :DH] * inv
    mean_h = a1[0, :, :DH] * inv
    dn = (((1,), (1,)), ((), ()))
    h = lax.dot_general(mean_l, wl_lo[...], dn,
                        preferred_element_type=jnp.float32)
    h = h + lax.dot_general(mean_h, wl_hi[...], dn,
                            preferred_element_type=jnp.float32)
    h = h + lax.dot_general(xr[...], wr[...], dn,
                            preferred_element_type=jnp.float32)
    h = jnp.maximum(h + bl[...], 0.0)
    o = lax.dot_general(h, wo[...], dn, preferred_element_type=jnp.float32)
    out[...] = o + bo[...]


def _tc_dense(agg, xr, wl_lo, wl_hi, wr, wo, bl, bo):
    blk = 1000
    return pl.pallas_call(
        _tc_body,
        grid=(N // blk,),
        in_specs=[
            pl.BlockSpec((1, blk, DH + CW), lambda i: (0, i, 0)),
            pl.BlockSpec((1, blk, DH + CW), lambda i: (1, i, 0)),
            pl.BlockSpec((blk, D), lambda i: (i, 0)),
            pl.BlockSpec((D, DH), lambda i: (0, 0)),
            pl.BlockSpec((D, DH), lambda i: (0, 0)),
            pl.BlockSpec((D, D), lambda i: (0, 0)),
            pl.BlockSpec((D, D), lambda i: (0, 0)),
            pl.BlockSpec((1, D), lambda i: (0, 0)),
            pl.BlockSpec((1, D), lambda i: (0, 0)),
        ],
        out_specs=pl.BlockSpec((blk, D), lambda i: (i, 0)),
        out_shape=jax.ShapeDtypeStruct((N, D), jnp.float32),
    )(agg, agg, xr, wl_lo, wl_hi, wr, wo, bl, bo)


def kernel(x_driver, x_race, x_circuit, edge_index_race_driver,
           edge_index_circuit_race, W_l0, b_l0, W_r0, W_l1, b_l1, W_r1,
           W_out, b_out):
    src = edge_index_circuit_race[0]
    dst = edge_index_circuit_race[1]
    pad = EPAD - E
    src_p = jnp.concatenate([src, jnp.zeros((pad,), jnp.int32)])
    dst_p = jnp.concatenate([dst, jnp.full((pad,), N, jnp.int32)])
    s2 = (src_p * 2).reshape(NS, NCH, CHUNK)
    agg = _sc_aggregate(s2, dst_p.reshape(NS, NCH, CHUNK),
                        x_circuit.reshape(2 * N, DH))
    out = _tc_dense(agg, x_race, W_l1[:, :DH], W_l1[:, DH:],
                    W_r1, W_out, b_l1.reshape(1, D), b_out.reshape(1, D))
    return out


# speedup vs baseline: 1.1578x; 1.1578x over previous
import functools

import jax
import jax.numpy as jnp
from jax import lax
from jax.experimental import pallas as pl
from jax.experimental.pallas import tpu as pltpu
from jax.experimental.pallas import tpu_sc as plsc

N = 10000
D = 128
DH = 64
CW = 16
E = 320000
NC = 2
NS = 16
CHUNK = 128
NCH = 157
EPAD = NS * NCH * CHUNK
NPAD = 10240
RPS = NPAD // NS
NBUF = 4
LAG = 2


@functools.partial(
    pl.kernel,
    out_type=jax.ShapeDtypeStruct((NC, NPAD, DH + CW), jnp.float32),
    mesh=plsc.VectorSubcoreMesh(core_axis_name="c", subcore_axis_name="s"),
    compiler_params=pltpu.CompilerParams(use_tc_tiling_on_sc=False),
    scratch_types=[
        pltpu.VMEM((NCH, CHUNK), jnp.int32),
        pltpu.VMEM((NCH, CHUNK), jnp.int32),
        pltpu.VMEM((NBUF, CHUNK, DH), jnp.float32),
        pltpu.VMEM((CHUNK, CW), jnp.float32),
        pltpu.VMEM((CHUNK, CW), jnp.float32),
        pltpu.VMEM_SHARED((NPAD, DH), jnp.float32),
        pltpu.VMEM_SHARED((NPAD, CW), jnp.float32),
        pltpu.SemaphoreType.DMA,
        pltpu.SemaphoreType.DMA,
        pltpu.SemaphoreType.DMA,
    ],
)
def _sc_aggregate(srcidx, dstidx, xflat, out_fc, src_v, dst_v, ring,
                  ones_v, zero_v, acc_f, acc_c, gsem, ssem, csem):
    cid = lax.axis_index("c")
    sid = lax.axis_index("s")

    zv = jnp.zeros((16,), jnp.float32)
    ov = jnp.ones((16,), jnp.float32)

    def zrow(i, carry):
        for k in range(DH // 16):
            ring[0, i, pl.ds(k * 16, 16)] = zv
        ones_v[i, pl.ds(0, 16)] = ov
        zero_v[i, pl.ds(0, 16)] = zv
        return carry

    lax.fori_loop(0, CHUNK, zrow, 0)
    for k in range(RPS // CHUNK):
        pltpu.sync_copy(ring.at[0],
                        acc_f.at[pl.ds(sid * RPS + k * CHUNK, CHUNK)])
        pltpu.sync_copy(zero_v,
                        acc_c.at[pl.ds(sid * RPS + k * CHUNK, CHUNK)])
    plsc.subcore_barrier()

    pltpu.sync_copy(srcidx.at[sid], src_v)
    pltpu.sync_copy(dstidx.at[sid], dst_v)
    xp = xflat.at[pl.ds(cid, 2 * N - 1)]

    def chunk(j, carry):
        b = lax.rem(j, NBUF)

        @pl.when(j < NCH)
        def _():
            @pl.when(j >= NBUF)
            def _():
                pltpu.make_async_copy(ring.at[b],
                                      acc_f.at[pl.ds(0, CHUNK)],
                                      ssem).wait()

            pltpu.async_copy(xp.at[src_v.at[j]], ring.at[b], gsem)

        d = j - LAG

        @pl.when(d >= 0)
        def _():
            db = lax.rem(d, NBUF)
            pltpu.make_async_copy(xp.at[pl.ds(0, CHUNK)], ring.at[db],
                                  gsem).wait()
            pltpu.async_copy(ring.at[db], acc_f.at[dst_v.at[d]], ssem,
                             add=True)

            @pl.when(cid == lax.rem(d, 2))
            def _():
                @pl.when(d >= 2)
                def _():
                    pltpu.make_async_copy(ones_v, acc_c.at[pl.ds(0, CHUNK)],
                                          csem).wait()

                pltpu.async_copy(ones_v, acc_c.at[dst_v.at[d]], csem,
                                 add=True)
        return carry

    lax.fori_loop(0, NCH + LAG, chunk, 0)

    for _ in range(NBUF):
        pltpu.make_async_copy(ring.at[0], acc_f.at[pl.ds(0, CHUNK)],
                              ssem).wait()
    pltpu.make_async_copy(ones_v, acc_c.at[pl.ds(0, CHUNK)], csem).wait()

    plsc.subcore_barrier()
    pltpu.sync_copy(acc_f.at[pl.ds(sid * RPS, RPS)],
                    out_fc.at[cid, pl.ds(sid * RPS, RPS), pl.ds(0, DH)])
    pltpu.sync_copy(acc_c.at[pl.ds(sid * RPS, RPS)],
                    out_fc.at[cid, pl.ds(sid * RPS, RPS), pl.ds(DH, CW)])


def _tc_body(a0, a1, xr, wl_lo, wl_hi, wr, wo, bl, bo, out):
    cnt = a0[0, :, DH:DH + 1] + a1[0, :, DH:DH + 1]
    inv = 1.0 / jnp.maximum(cnt, 1.0)
    mean_l = a0[0, :, :DH] * inv
    mean_h = a1[0, :, :DH] * inv
    dn = (((1,), (1,)), ((), ()))
    h = lax.dot_general(mean_l, wl_lo[...], dn,
                        preferred_element_type=jnp.float32)
    h = h + lax.dot_general(mean_h, wl_hi[...], dn,
                            preferred_element_type=jnp.float32)
    h = h + lax.dot_general(xr[...], wr[...], dn,
                            preferred_element_type=jnp.float32)
    h = jnp.maximum(h + bl[...], 0.0)
    o = lax.dot_general(h, wo[...], dn, preferred_element_type=jnp.float32)
    out[...] = o + bo[...]


def _tc_dense(agg, xr, wl_lo, wl_hi, wr, wo, bl, bo):
    blk = 1000
    return pl.pallas_call(
        _tc_body,
        grid=(N // blk,),
        in_specs=[
            pl.BlockSpec((1, blk, DH + CW), lambda i: (0, i, 0)),
            pl.BlockSpec((1, blk, DH + CW), lambda i: (1, i, 0)),
            pl.BlockSpec((blk, D), lambda i: (i, 0)),
            pl.BlockSpec((D, DH), lambda i: (0, 0)),
            pl.BlockSpec((D, DH), lambda i: (0, 0)),
            pl.BlockSpec((D, D), lambda i: (0, 0)),
            pl.BlockSpec((D, D), lambda i: (0, 0)),
            pl.BlockSpec((1, D), lambda i: (0, 0)),
            pl.BlockSpec((1, D), lambda i: (0, 0)),
        ],
        out_specs=pl.BlockSpec((blk, D), lambda i: (i, 0)),
        out_shape=jax.ShapeDtypeStruct((N, D), jnp.float32),
    )(agg, agg, xr, wl_lo, wl_hi, wr, wo, bl, bo)


def kernel(x_driver, x_race, x_circuit, edge_index_race_driver,
           edge_index_circuit_race, W_l0, b_l0, W_r0, W_l1, b_l1, W_r1,
           W_out, b_out):
    src = edge_index_circuit_race[0]
    dst = edge_index_circuit_race[1]
    pad = EPAD - E
    src_p = jnp.concatenate([src, jnp.zeros((pad,), jnp.int32)])
    dst_p = jnp.concatenate([dst, jnp.full((pad,), N, jnp.int32)])
    s2 = (src_p * 2).reshape(NS, NCH, CHUNK)
    agg = _sc_aggregate(s2, dst_p.reshape(NS, NCH, CHUNK),
                        x_circuit.reshape(2 * N, DH))
    out = _tc_dense(agg, x_race, W_l1[:, :DH], W_l1[:, DH:],
                    W_r1, W_out, b_l1.reshape(1, D), b_out.reshape(1, D))
    return out


# speedup vs baseline: 1.1581x; 1.0003x over previous
import functools

import jax
import jax.numpy as jnp
from jax import lax
from jax.experimental import pallas as pl
from jax.experimental.pallas import tpu as pltpu
from jax.experimental.pallas import tpu_sc as plsc

N = 10000
D = 128
DH = 64
CW = 16
E = 320000
NC = 2
NS = 16
CHUNK = 128
NCH = 157
EPAD = NS * NCH * CHUNK
NPAD = 10240
RPS = NPAD // NS
NBUF = 4
LAG = 3


@functools.partial(
    pl.kernel,
    out_type=jax.ShapeDtypeStruct((NC, NPAD, DH + CW), jnp.float32),
    mesh=plsc.VectorSubcoreMesh(core_axis_name="c", subcore_axis_name="s"),
    compiler_params=pltpu.CompilerParams(use_tc_tiling_on_sc=False),
    scratch_types=[
        pltpu.VMEM((NCH, CHUNK), jnp.int32),
        pltpu.VMEM((NCH, CHUNK), jnp.int32),
        pltpu.VMEM((NBUF, CHUNK, DH), jnp.float32),
        pltpu.VMEM((CHUNK, CW), jnp.float32),
        pltpu.VMEM((CHUNK, CW), jnp.float32),
        pltpu.VMEM_SHARED((NPAD, DH), jnp.float32),
        pltpu.VMEM_SHARED((NPAD, CW), jnp.float32),
        pltpu.SemaphoreType.DMA,
        pltpu.SemaphoreType.DMA,
        pltpu.SemaphoreType.DMA,
    ],
)
def _sc_aggregate(srcidx, dstidx, xflat, out_fc, src_v, dst_v, ring,
                  ones_v, zero_v, acc_f, acc_c, gsem, ssem, csem):
    cid = lax.axis_index("c")
    sid = lax.axis_index("s")

    zv = jnp.zeros((16,), jnp.float32)
    ov = jnp.ones((16,), jnp.float32)

    def zrow(i, carry):
        for k in range(DH // 16):
            ring[0, i, pl.ds(k * 16, 16)] = zv
        ones_v[i, pl.ds(0, 16)] = ov
        zero_v[i, pl.ds(0, 16)] = zv
        return carry

    lax.fori_loop(0, CHUNK, zrow, 0)
    for k in range(RPS // CHUNK):
        pltpu.sync_copy(ring.at[0],
                        acc_f.at[pl.ds(sid * RPS + k * CHUNK, CHUNK)])
        pltpu.sync_copy(zero_v,
                        acc_c.at[pl.ds(sid * RPS + k * CHUNK, CHUNK)])
    plsc.subcore_barrier()

    pltpu.sync_copy(srcidx.at[sid], src_v)
    pltpu.sync_copy(dstidx.at[sid], dst_v)
    xp = xflat.at[pl.ds(cid, 2 * N - 1)]

    def chunk(j, carry):
        b = lax.rem(j, NBUF)

        @pl.when(j < NCH)
        def _():
            @pl.when(j >= NBUF)
            def _():
                pltpu.make_async_copy(ring.at[b],
                                      acc_f.at[pl.ds(0, CHUNK)],
                                      ssem).wait()

            pltpu.async_copy(xp.at[src_v.at[j]], ring.at[b], gsem)

        d = j - LAG

        @pl.when(d >= 0)
        def _():
            db = lax.rem(d, NBUF)
            pltpu.make_async_copy(xp.at[pl.ds(0, CHUNK)], ring.at[db],
                                  gsem).wait()
            pltpu.async_copy(ring.at[db], acc_f.at[dst_v.at[d]], ssem,
                             add=True)

            @pl.when(cid == lax.rem(d, 2))
            def _():
                @pl.when(d >= 2)
                def _():
                    pltpu.make_async_copy(ones_v, acc_c.at[pl.ds(0, CHUNK)],
                                          csem).wait()

                pltpu.async_copy(ones_v, acc_c.at[dst_v.at[d]], csem,
                                 add=True)
        return carry

    lax.fori_loop(0, NCH + LAG, chunk, 0)

    for _ in range(NBUF):
        pltpu.make_async_copy(ring.at[0], acc_f.at[pl.ds(0, CHUNK)],
                              ssem).wait()
    pltpu.make_async_copy(ones_v, acc_c.at[pl.ds(0, CHUNK)], csem).wait()

    plsc.subcore_barrier()
    pltpu.sync_copy(acc_f.at[pl.ds(sid * RPS, RPS)],
                    out_fc.at[cid, pl.ds(sid * RPS, RPS), pl.ds(0, DH)])
    pltpu.sync_copy(acc_c.at[pl.ds(sid * RPS, RPS)],
                    out_fc.at[cid, pl.ds(sid * RPS, RPS), pl.ds(DH, CW)])


def _tc_body(a0, a1, xr, wl_lo, wl_hi, wr, wo, bl, bo, out):
    cnt = a0[0, :, DH:DH + 1] + a1[0, :, DH:DH + 1]
    inv = 1.0 / jnp.maximum(cnt, 1.0)
    mean_l = a0[0, :, :DH] * inv
    mean_h = a1[0, :, :DH] * inv
    dn = (((1,), (1,)), ((), ()))
    h = lax.dot_general(mean_l, wl_lo[...], dn,
                        preferred_element_type=jnp.float32)
    h = h + lax.dot_general(mean_h, wl_hi[...], dn,
                            preferred_element_type=jnp.float32)
    h = h + lax.dot_general(xr[...], wr[...], dn,
                            preferred_element_type=jnp.float32)
    h = jnp.maximum(h + bl[...], 0.0)
    o = lax.dot_general(h, wo[...], dn, preferred_element_type=jnp.float32)
    out[...] = o + bo[...]


def _tc_dense(agg, xr, wl_lo, wl_hi, wr, wo, bl, bo):
    blk = 1000
    return pl.pallas_call(
        _tc_body,
        grid=(N // blk,),
        in_specs=[
            pl.BlockSpec((1, blk, DH + CW), lambda i: (0, i, 0)),
            pl.BlockSpec((1, blk, DH + CW), lambda i: (1, i, 0)),
            pl.BlockSpec((blk, D), lambda i: (i, 0)),
            pl.BlockSpec((D, DH), lambda i: (0, 0)),
            pl.BlockSpec((D, DH), lambda i: (0, 0)),
            pl.BlockSpec((D, D), lambda i: (0, 0)),
            pl.BlockSpec((D, D), lambda i: (0, 0)),
            pl.BlockSpec((1, D), lambda i: (0, 0)),
            pl.BlockSpec((1, D), lambda i: (0, 0)),
        ],
        out_specs=pl.BlockSpec((blk, D), lambda i: (i, 0)),
        out_shape=jax.ShapeDtypeStruct((N, D), jnp.float32),
    )(agg, agg, xr, wl_lo, wl_hi, wr, wo, bl, bo)


def kernel(x_driver, x_race, x_circuit, edge_index_race_driver,
           edge_index_circuit_race, W_l0, b_l0, W_r0, W_l1, b_l1, W_r1,
           W_out, b_out):
    src = edge_index_circuit_race[0]
    dst = edge_index_circuit_race[1]
    pad = EPAD - E
    src_p = jnp.concatenate([src, jnp.zeros((pad,), jnp.int32)])
    dst_p = jnp.concatenate([dst, jnp.full((pad,), N, jnp.int32)])
    s2 = (src_p * 2).reshape(NS, NCH, CHUNK)
    agg = _sc_aggregate(s2, dst_p.reshape(NS, NCH, CHUNK),
                        x_circuit.reshape(2 * N, DH))
    out = _tc_dense(agg, x_race, W_l1[:, :DH], W_l1[:, DH:],
                    W_r1, W_out, b_l1.reshape(1, D), b_out.reshape(1, D))
    return out


# speedup vs baseline: 1.1584x; 1.0003x over previous
import functools

import jax
import jax.numpy as jnp
from jax import lax
from jax.experimental import pallas as pl
from jax.experimental.pallas import tpu as pltpu
from jax.experimental.pallas import tpu_sc as plsc

N = 10000
D = 128
DH = 64
CW = 16
E = 320000
NC = 2
NS = 16
CHUNK = 128
NCH = 157
EPAD = NS * NCH * CHUNK
NPAD = 10240
RPS = NPAD // NS
NBUF = 4
LAG = 2


@functools.partial(
    pl.kernel,
    out_type=jax.ShapeDtypeStruct((NC, NPAD, DH + CW), jnp.float32),
    mesh=plsc.VectorSubcoreMesh(core_axis_name="c", subcore_axis_name="s"),
    compiler_params=pltpu.CompilerParams(use_tc_tiling_on_sc=False),
    scratch_types=[
        pltpu.VMEM((NCH, CHUNK), jnp.int32),
        pltpu.VMEM((NCH, CHUNK), jnp.int32),
        pltpu.VMEM((NBUF, CHUNK, DH), jnp.float32),
        pltpu.VMEM((CHUNK, CW), jnp.float32),
        pltpu.VMEM((CHUNK, CW), jnp.float32),
        pltpu.VMEM_SHARED((NPAD, DH), jnp.float32),
        pltpu.VMEM_SHARED((NPAD, CW), jnp.float32),
        pltpu.SemaphoreType.DMA,
        pltpu.SemaphoreType.DMA,
        pltpu.SemaphoreType.DMA,
    ],
)
def _sc_aggregate(srcidx, dstidx, xflat, out_fc, src_v, dst_v, ring,
                  ones_v, zero_v, acc_f, acc_c, gsem, ssem, csem):
    cid = lax.axis_index("c")
    sid = lax.axis_index("s")

    zv = jnp.zeros((16,), jnp.float32)
    ov = jnp.ones((16,), jnp.float32)

    def zrow(i, carry):
        for k in range(DH // 16):
            ring[0, i, pl.ds(k * 16, 16)] = zv
        ones_v[i, pl.ds(0, 16)] = ov
        zero_v[i, pl.ds(0, 16)] = zv
        return carry

    lax.fori_loop(0, CHUNK, zrow, 0)
    for k in range(RPS // CHUNK):
        pltpu.sync_copy(ring.at[0],
                        acc_f.at[pl.ds(sid * RPS + k * CHUNK, CHUNK)])
        pltpu.sync_copy(zero_v,
                        acc_c.at[pl.ds(sid * RPS + k * CHUNK, CHUNK)])
    plsc.subcore_barrier()

    pltpu.sync_copy(srcidx.at[sid], src_v)
    pltpu.sync_copy(dstidx.at[sid], dst_v)
    xp = xflat.at[pl.ds(cid, 2 * N - 1)]

    def chunk(j, carry):
        b = lax.rem(j, NBUF)

        @pl.when(j < NCH)
        def _():
            @pl.when(j >= NBUF)
            def _():
                pltpu.make_async_copy(ring.at[b],
                                      acc_f.at[pl.ds(0, CHUNK)],
                                      ssem).wait()

            pltpu.async_copy(xp.at[src_v.at[j]], ring.at[b], gsem)

        d = j - LAG

        @pl.when(d >= 0)
        def _():
            db = lax.rem(d, NBUF)
            pltpu.make_async_copy(xp.at[pl.ds(0, CHUNK)], ring.at[db],
                                  gsem).wait()
            pltpu.async_copy(ring.at[db], acc_f.at[dst_v.at[d]], ssem,
                             add=True)

            @pl.when(cid == lax.rem(d, 2))
            def _():
                @pl.when(d >= 2)
                def _():
                    pltpu.make_async_copy(ones_v, acc_c.at[pl.ds(0, CHUNK)],
                                          csem).wait()

                pltpu.async_copy(ones_v, acc_c.at[dst_v.at[d]], csem,
                                 add=True)
        return carry

    lax.fori_loop(0, NCH + LAG, chunk, 0)

    for _ in range(NBUF):
        pltpu.make_async_copy(ring.at[0], acc_f.at[pl.ds(0, CHUNK)],
                              ssem).wait()
    pltpu.make_async_copy(ones_v, acc_c.at[pl.ds(0, CHUNK)], csem).wait()

    plsc.subcore_barrier()
    pltpu.sync_copy(acc_f.at[pl.ds(sid * RPS, RPS)],
                    out_fc.at[cid, pl.ds(sid * RPS, RPS), pl.ds(0, DH)])
    pltpu.sync_copy(acc_c.at[pl.ds(sid * RPS, RPS)],
                    out_fc.at[cid, pl.ds(sid * RPS, RPS), pl.ds(DH, CW)])


def _tc_body(a0, a1, xr, wl_lo, wl_hi, wr, wo, bl, bo, out):
    cnt = a0[0, :, DH:DH + 1] + a1[0, :, DH:DH + 1]
    inv = 1.0 / jnp.maximum(cnt, 1.0)
    mean_l = a0[0, :, :DH] * inv
    mean_h = a1[0, :, :DH] * inv
    dn = (((1,), (1,)), ((), ()))
    h = lax.dot_general(mean_l, wl_lo[...], dn,
                        preferred_element_type=jnp.float32)
    h = h + lax.dot_general(mean_h, wl_hi[...], dn,
                            preferred_element_type=jnp.float32)
    h = h + lax.dot_general(xr[...], wr[...], dn,
                            preferred_element_type=jnp.float32)
    h = jnp.maximum(h + bl[...], 0.0)
    o = lax.dot_general(h, wo[...], dn, preferred_element_type=jnp.float32)
    out[...] = o + bo[...]


def _tc_dense(agg, xr, wl_lo, wl_hi, wr, wo, bl, bo):
    blk = 1000
    return pl.pallas_call(
        _tc_body,
        grid=(N // blk,),
        in_specs=[
            pl.BlockSpec((1, blk, DH + CW), lambda i: (0, i, 0)),
            pl.BlockSpec((1, blk, DH + CW), lambda i: (1, i, 0)),
            pl.BlockSpec((blk, D), lambda i: (i, 0)),
            pl.BlockSpec((D, DH), lambda i: (0, 0)),
            pl.BlockSpec((D, DH), lambda i: (0, 0)),
            pl.BlockSpec((D, D), lambda i: (0, 0)),
            pl.BlockSpec((D, D), lambda i: (0, 0)),
            pl.BlockSpec((1, D), lambda i: (0, 0)),
            pl.BlockSpec((1, D), lambda i: (0, 0)),
        ],
        out_specs=pl.BlockSpec((blk, D), lambda i: (i, 0)),
        out_shape=jax.ShapeDtypeStruct((N, D), jnp.float32),
    )(agg, agg, xr, wl_lo, wl_hi, wr, wo, bl, bo)


def kernel(x_driver, x_race, x_circuit, edge_index_race_driver,
           edge_index_circuit_race, W_l0, b_l0, W_r0, W_l1, b_l1, W_r1,
           W_out, b_out):
    src = edge_index_circuit_race[0]
    dst = edge_index_circuit_race[1]
    pad = EPAD - E
    src_p = jnp.concatenate([src, jnp.zeros((pad,), jnp.int32)])
    dst_p = jnp.concatenate([dst, jnp.full((pad,), N, jnp.int32)])
    s2 = (src_p * 2).reshape(NS, NCH, CHUNK)
    agg = _sc_aggregate(s2, dst_p.reshape(NS, NCH, CHUNK),
                        x_circuit.reshape(2 * N, DH))
    out = _tc_dense(agg, x_race, W_l1[:, :DH], W_l1[:, DH:],
                    W_r1, W_out, b_l1.reshape(1, D), b_out.reshape(1, D))
    return out


# speedup vs baseline: 1.1632x; 1.0041x over previous
import functools

import jax
import jax.numpy as jnp
from jax import lax
from jax.experimental import pallas as pl
from jax.experimental.pallas import tpu as pltpu
from jax.experimental.pallas import tpu_sc as plsc

N = 10000
D = 128
DH = 64
CW = 16
E = 320000
NC = 2
NS = 16
CHUNK = 64
NCH = 314
EPAD = NS * NCH * CHUNK
NPAD = 10240
RPS = NPAD // NS
NBUF = 8
LAG = 4


@functools.partial(
    pl.kernel,
    out_type=jax.ShapeDtypeStruct((NC, NPAD, DH + CW), jnp.float32),
    mesh=plsc.VectorSubcoreMesh(core_axis_name="c", subcore_axis_name="s"),
    compiler_params=pltpu.CompilerParams(use_tc_tiling_on_sc=False),
    scratch_types=[
        pltpu.VMEM((NCH, CHUNK), jnp.int32),
        pltpu.VMEM((NCH, CHUNK), jnp.int32),
        pltpu.VMEM((NBUF, CHUNK, DH), jnp.float32),
        pltpu.VMEM((CHUNK, CW), jnp.float32),
        pltpu.VMEM((CHUNK, CW), jnp.float32),
        pltpu.VMEM_SHARED((NPAD, DH), jnp.float32),
        pltpu.VMEM_SHARED((NPAD, CW), jnp.float32),
        pltpu.SemaphoreType.DMA,
        pltpu.SemaphoreType.DMA,
        pltpu.SemaphoreType.DMA,
    ],
)
def _sc_aggregate(srcidx, dstidx, xflat, out_fc, src_v, dst_v, ring,
                  ones_v, zero_v, acc_f, acc_c, gsem, ssem, csem):
    cid = lax.axis_index("c")
    sid = lax.axis_index("s")

    zv = jnp.zeros((16,), jnp.float32)
    ov = jnp.ones((16,), jnp.float32)

    def zrow(i, carry):
        for k in range(DH // 16):
            ring[0, i, pl.ds(k * 16, 16)] = zv
        ones_v[i, pl.ds(0, 16)] = ov
        zero_v[i, pl.ds(0, 16)] = zv
        return carry

    lax.fori_loop(0, CHUNK, zrow, 0)
    for k in range(RPS // CHUNK):
        pltpu.sync_copy(ring.at[0],
                        acc_f.at[pl.ds(sid * RPS + k * CHUNK, CHUNK)])
        pltpu.sync_copy(zero_v,
                        acc_c.at[pl.ds(sid * RPS + k * CHUNK, CHUNK)])
    plsc.subcore_barrier()

    pltpu.sync_copy(srcidx.at[sid], src_v)
    pltpu.sync_copy(dstidx.at[sid], dst_v)
    xp = xflat.at[pl.ds(cid, 2 * N - 1)]

    def chunk(j, carry):
        b = lax.rem(j, NBUF)

        @pl.when(j < NCH)
        def _():
            @pl.when(j >= NBUF)
            def _():
                pltpu.make_async_copy(ring.at[b],
                                      acc_f.at[pl.ds(0, CHUNK)],
                                      ssem).wait()

            pltpu.async_copy(xp.at[src_v.at[j]], ring.at[b], gsem)

        d = j - LAG

        @pl.when(d >= 0)
        def _():
            db = lax.rem(d, NBUF)
            pltpu.make_async_copy(xp.at[pl.ds(0, CHUNK)], ring.at[db],
                                  gsem).wait()
            pltpu.async_copy(ring.at[db], acc_f.at[dst_v.at[d]], ssem,
                             add=True)

            @pl.when(cid == lax.rem(d, 2))
            def _():
                @pl.when(d >= 2)
                def _():
                    pltpu.make_async_copy(ones_v, acc_c.at[pl.ds(0, CHUNK)],
                                          csem).wait()

                pltpu.async_copy(ones_v, acc_c.at[dst_v.at[d]], csem,
                                 add=True)
        return carry

    lax.fori_loop(0, NCH + LAG, chunk, 0)

    for _ in range(NBUF):
        pltpu.make_async_copy(ring.at[0], acc_f.at[pl.ds(0, CHUNK)],
                              ssem).wait()
    pltpu.make_async_copy(ones_v, acc_c.at[pl.ds(0, CHUNK)], csem).wait()

    plsc.subcore_barrier()
    pltpu.sync_copy(acc_f.at[pl.ds(sid * RPS, RPS)],
                    out_fc.at[cid, pl.ds(sid * RPS, RPS), pl.ds(0, DH)])
    pltpu.sync_copy(acc_c.at[pl.ds(sid * RPS, RPS)],
                    out_fc.at[cid, pl.ds(sid * RPS, RPS), pl.ds(DH, CW)])


def _tc_body(a0, a1, xr, wl_lo, wl_hi, wr, wo, bl, bo, out):
    cnt = a0[0, :, DH:DH + 1] + a1[0, :, DH:DH + 1]
    inv = 1.0 / jnp.maximum(cnt, 1.0)
    mean_l = a0[0, :, :DH] * inv
    mean_h = a1[0, :, :DH] * inv
    dn = (((1,), (1,)), ((), ()))
    h = lax.dot_general(mean_l, wl_lo[...], dn,
                        preferred_element_type=jnp.float32)
    h = h + lax.dot_general(mean_h, wl_hi[...], dn,
                            preferred_element_type=jnp.float32)
    h = h + lax.dot_general(xr[...], wr[...], dn,
                            preferred_element_type=jnp.float32)
    h = jnp.maximum(h + bl[...], 0.0)
    o = lax.dot_general(h, wo[...], dn, preferred_element_type=jnp.float32)
    out[...] = o + bo[...]


def _tc_dense(agg, xr, wl_lo, wl_hi, wr, wo, bl, bo):
    blk = 1000
    return pl.pallas_call(
        _tc_body,
        grid=(N // blk,),
        in_specs=[
            pl.BlockSpec((1, blk, DH + CW), lambda i: (0, i, 0)),
            pl.BlockSpec((1, blk, DH + CW), lambda i: (1, i, 0)),
            pl.BlockSpec((blk, D), lambda i: (i, 0)),
            pl.BlockSpec((D, DH), lambda i: (0, 0)),
            pl.BlockSpec((D, DH), lambda i: (0, 0)),
            pl.BlockSpec((D, D), lambda i: (0, 0)),
            pl.BlockSpec((D, D), lambda i: (0, 0)),
            pl.BlockSpec((1, D), lambda i: (0, 0)),
            pl.BlockSpec((1, D), lambda i: (0, 0)),
        ],
        out_specs=pl.BlockSpec((blk, D), lambda i: (i, 0)),
        out_shape=jax.ShapeDtypeStruct((N, D), jnp.float32),
    )(agg, agg, xr, wl_lo, wl_hi, wr, wo, bl, bo)


def kernel(x_driver, x_race, x_circuit, edge_index_race_driver,
           edge_index_circuit_race, W_l0, b_l0, W_r0, W_l1, b_l1, W_r1,
           W_out, b_out):
    src = edge_index_circuit_race[0]
    dst = edge_index_circuit_race[1]
    pad = EPAD - E
    src_p = jnp.concatenate([src, jnp.zeros((pad,), jnp.int32)])
    dst_p = jnp.concatenate([dst, jnp.full((pad,), N, jnp.int32)])
    s2 = (src_p * 2).reshape(NS, NCH, CHUNK)
    agg = _sc_aggregate(s2, dst_p.reshape(NS, NCH, CHUNK),
                        x_circuit.reshape(2 * N, DH))
    out = _tc_dense(agg, x_race, W_l1[:, :DH], W_l1[:, DH:],
                    W_r1, W_out, b_l1.reshape(1, D), b_out.reshape(1, D))
    return out
